# baseline (speedup 1.0000x reference)
import jax
import jax.numpy as jnp
from jax.experimental import pallas as pl
from jax.experimental.pallas import tpu as pltpu

_NS = 2
_NO = 2


def _make_dequant_kernel(n, m, hw, k, d):
    def body(s_hbm, c_hbm, o_hbm, s_buf, c_buf, o_buf, s_sem, c_sem, o_sem):
        def s_copy(ni, mi):
            return pltpu.make_async_copy(
                s_hbm.at[ni, mi], s_buf.at[ni % _NS, mi], s_sem.at[ni % _NS, mi])

        def o_copy(ni):
            return pltpu.make_async_copy(
                o_buf.at[ni % _NO], o_hbm.at[ni], o_sem.at[ni % _NO])

        pltpu.make_async_copy(c_hbm, c_buf, c_sem).start()
        for ni in range(_NS):
            for mi in range(m):
                s_copy(ni, mi).start()
        pltpu.make_async_copy(c_hbm, c_buf, c_sem).wait()

        for ni in range(n):
            if ni >= _NO:
                o_copy(ni - _NO).wait()
            for mi in range(m):
                s_copy(ni, mi).wait()
                c = c_buf[mi].astype(jnp.bfloat16)
                s = s_buf[ni % _NS, mi].astype(jnp.bfloat16)
                o_buf[ni % _NO, mi] = jax.lax.dot_general(
                    c, s, (((0,), (1,)), ((), ())),
                    preferred_element_type=jnp.float32)
                if ni + _NS < n:
                    s_copy(ni + _NS, mi).start()
            o_copy(ni).start()

        for ni in range(n - _NO, n):
            o_copy(ni).wait()

    return body


def kernel(sample, codebook):
    n, m, h, w, k = sample.shape
    d = codebook.shape[-1]
    hw = h * w
    s = sample.reshape(n, m, hw, k)
    out = pl.pallas_call(
        _make_dequant_kernel(n, m, hw, k, d),
        in_specs=[
            pl.BlockSpec(memory_space=pl.ANY),
            pl.BlockSpec(memory_space=pl.ANY),
        ],
        out_specs=pl.BlockSpec(memory_space=pl.ANY),
        out_shape=jax.ShapeDtypeStruct((n, m, d, hw), jnp.float32),
        scratch_shapes=[
            pltpu.VMEM((_NS, m, hw, k), jnp.float32),
            pltpu.VMEM((m, k, d), jnp.float32),
            pltpu.VMEM((_NO, m, d, hw), jnp.float32),
            pltpu.SemaphoreType.DMA((_NS, m)),
            pltpu.SemaphoreType.DMA,
            pltpu.SemaphoreType.DMA((_NO,)),
        ],
    )(s, codebook)
    return out.reshape(n, m * d, h, w)
